# baseline (speedup 1.0000x reference)
import functools

import jax
import jax.numpy as jnp
from jax.experimental import pallas as pl
from jax.experimental.pallas import tpu as pltpu

H = 64
L = 7
ADJ_TYPES = ['0_0', '1_0', '0_1', '1_1', '2_1', '1_2']
HEAD_BLOCK = 2000


def _silu(x):
    return x * jax.lax.logistic(x)


def _head_body(x0_ref, loc_ref, y_ref, pw1_ref, pb1_ref, pw2_ref, pb2_ref,
               postw1_ref, postb1_ref, postw2_ref, postb2_ref,
               loss_ref, lsum_ref, cnt_ref):
    x = x0_ref[...]
    h = _silu(x @ pw1_ref[...] + pb1_ref[...])
    h = h @ pw2_ref[...] + pb2_ref[...]
    o = _silu(h @ postw1_ref[...] + postb1_ref[...])
    o = o @ postw2_ref[...] + postb2_ref[...]
    o = o + loc_ref[...]
    t = y_ref[...]
    mask = ~jnp.isnan(t)
    diff2 = (o - t) ** 2
    loss = jnp.where(mask, diff2, 0.0)
    loss_ref[...] = loss

    @pl.when(pl.program_id(0) == 0)
    def _init():
        lsum_ref[...] = jnp.zeros_like(lsum_ref)
        cnt_ref[...] = jnp.zeros_like(cnt_ref)

    lsum_ref[...] += jnp.sum(loss).reshape(1, 1)
    cnt_ref[...] += jnp.sum(mask.astype(jnp.float32)).reshape(1, 1)


def _head(x0, loc, y, pre_W1, pre_b1, pre_W2, pre_b2,
          post_W1, post_b1, post_W2, post_b2):
    n0 = x0.shape[0]
    grid = n0 // HEAD_BLOCK
    row = lambda i: (i, 0)
    fixed = lambda i: (0, 0)
    loss, lsum, cnt = pl.pallas_call(
        _head_body,
        grid=(grid,),
        in_specs=[
            pl.BlockSpec((HEAD_BLOCK, H), row),
            pl.BlockSpec((HEAD_BLOCK, 3), row),
            pl.BlockSpec((HEAD_BLOCK, 3), row),
            pl.BlockSpec((H, H), fixed),
            pl.BlockSpec((1, H), fixed),
            pl.BlockSpec((H, H), fixed),
            pl.BlockSpec((1, H), fixed),
            pl.BlockSpec((H, H), fixed),
            pl.BlockSpec((1, H), fixed),
            pl.BlockSpec((H, 3), fixed),
            pl.BlockSpec((1, 3), fixed),
        ],
        out_specs=[
            pl.BlockSpec((HEAD_BLOCK, 3), row),
            pl.BlockSpec((1, 1), fixed),
            pl.BlockSpec((1, 1), fixed),
        ],
        out_shape=[
            jax.ShapeDtypeStruct((n0, 3), jnp.float32),
            jax.ShapeDtypeStruct((1, 1), jnp.float32),
            jax.ShapeDtypeStruct((1, 1), jnp.float32),
        ],
    )(x0, loc, y,
      pre_W1, pre_b1.reshape(1, H), pre_W2, pre_b2.reshape(1, H),
      post_W1, post_b1.reshape(1, H), post_W2, post_b2.reshape(1, 3))
    backprop = lsum[0, 0] / jnp.maximum(cnt[0, 0], 1.0)
    return backprop, loss


def kernel(loc, vel, charges, y, x_0, x_0_batch, x_1, x_1_batch, x_2,
           x_2_batch, adj_0_0, adj_0_1, adj_1_1, adj_1_2, step, emb_W, emb_b,
           W_self, W_adj, pre_W1, pre_b1, pre_W2, pre_b2, post_W1, post_b1,
           post_W2, post_b2):
    feat_in = jnp.concatenate(
        [jnp.linalg.norm(vel, axis=-1, keepdims=True), charges], axis=-1)
    g0 = x_0 + x_0_batch[:, None] * 5
    g1 = x_1 + x_1_batch[:, None] * 5
    g2 = x_2 + x_2_batch[:, None] * 5
    x = {'0': feat_in[g0[:, 0]],
         '1': (feat_in[g1[:, 0]] + feat_in[g1[:, 1]]) / 2.0,
         '2': (feat_in[g2[:, 0]] + feat_in[g2[:, 1]] + feat_in[g2[:, 2]]) / 3.0}
    x = {d: f @ emb_W + emb_b for d, f in x.items()}
    adj = {'0_0': adj_0_0, '0_1': adj_0_1, '1_1': adj_1_1, '1_2': adj_1_2}
    adj['1_0'] = adj['0_1'][jnp.array([1, 0])]
    adj['2_1'] = adj['1_2'][jnp.array([1, 0])]
    nseg = {'0': x['0'].shape[0], '1': x['1'].shape[0], '2': x['2'].shape[0]}
    needed = [(0, 1, 2)] * 5 + [(0, 1), (0,)]
    for l in range(L):
        new = {}
        for d in needed[l]:
            agg = x[str(d)] @ W_self[l, d]
            for ai, at in enumerate(ADJ_TYPES):
                s, t = at.split('_')
                if int(t) == d:
                    e = adj[at]
                    seg = jax.ops.segment_sum(x[s][e[0]], e[1],
                                              num_segments=nseg[str(d)])
                    agg = agg + seg @ W_adj[l, ai]
            new[str(d)] = _silu(agg)
        x = new
    return _head(x['0'], loc, y.reshape(-1, 3),
                 pre_W1[0], pre_b1[0], pre_W2[0], pre_b2[0],
                 post_W1, post_b1, post_W2, post_b2)


# speedup vs baseline: 1.0331x; 1.0331x over previous
import functools

import jax
import jax.numpy as jnp
from jax.experimental import pallas as pl
from jax.experimental.pallas import tpu as pltpu

H = 64
L = 7
ADJ_TYPES = ['0_0', '1_0', '0_1', '1_1', '2_1', '1_2']
HEAD_BLOCK = 2000


def _silu(x):
    return x * jax.lax.logistic(x)


def _head_body(x0_ref, loc_ref, y_ref, pw1_ref, pb1_ref, pw2_ref, pb2_ref,
               postw1_ref, postb1_ref, postw2_ref, postb2_ref,
               loss_ref, lsum_ref, cnt_ref):
    x = x0_ref[...]
    h = _silu(x @ pw1_ref[...] + pb1_ref[...])
    h = h @ pw2_ref[...] + pb2_ref[...]
    o = _silu(h @ postw1_ref[...] + postb1_ref[...])
    o = o @ postw2_ref[...] + postb2_ref[...]
    o = o + loc_ref[...]
    t = y_ref[...]
    mask = ~jnp.isnan(t)
    diff2 = (o - t) ** 2
    loss = jnp.where(mask, diff2, 0.0)
    loss_ref[...] = loss

    @pl.when(pl.program_id(0) == 0)
    def _init():
        lsum_ref[...] = jnp.zeros_like(lsum_ref)
        cnt_ref[...] = jnp.zeros_like(cnt_ref)

    lsum_ref[...] += jnp.sum(loss).reshape(1, 1)
    cnt_ref[...] += jnp.sum(mask.astype(jnp.float32)).reshape(1, 1)


def _head(x0, loc, y, pre_W1, pre_b1, pre_W2, pre_b2,
          post_W1, post_b1, post_W2, post_b2):
    n0 = x0.shape[0]
    grid = n0 // HEAD_BLOCK
    row = lambda i: (i, 0)
    fixed = lambda i: (0, 0)
    loss, lsum, cnt = pl.pallas_call(
        _head_body,
        grid=(grid,),
        in_specs=[
            pl.BlockSpec((HEAD_BLOCK, H), row),
            pl.BlockSpec((HEAD_BLOCK, 3), row),
            pl.BlockSpec((HEAD_BLOCK, 3), row),
            pl.BlockSpec((H, H), fixed),
            pl.BlockSpec((1, H), fixed),
            pl.BlockSpec((H, H), fixed),
            pl.BlockSpec((1, H), fixed),
            pl.BlockSpec((H, H), fixed),
            pl.BlockSpec((1, H), fixed),
            pl.BlockSpec((H, 3), fixed),
            pl.BlockSpec((1, 3), fixed),
        ],
        out_specs=[
            pl.BlockSpec((HEAD_BLOCK, 3), row),
            pl.BlockSpec((1, 1), fixed),
            pl.BlockSpec((1, 1), fixed),
        ],
        out_shape=[
            jax.ShapeDtypeStruct((n0, 3), jnp.float32),
            jax.ShapeDtypeStruct((1, 1), jnp.float32),
            jax.ShapeDtypeStruct((1, 1), jnp.float32),
        ],
    )(x0, loc, y,
      pre_W1, pre_b1.reshape(1, H), pre_W2, pre_b2.reshape(1, H),
      post_W1, post_b1.reshape(1, H), post_W2, post_b2.reshape(1, 3))
    backprop = lsum[0, 0] / jnp.maximum(cnt[0, 0], 1.0)
    return backprop, loss


def kernel(loc, vel, charges, y, x_0, x_0_batch, x_1, x_1_batch, x_2,
           x_2_batch, adj_0_0, adj_0_1, adj_1_1, adj_1_2, step, emb_W, emb_b,
           W_self, W_adj, pre_W1, pre_b1, pre_W2, pre_b2, post_W1, post_b1,
           post_W2, post_b2):
    feat_in = jnp.concatenate(
        [jnp.linalg.norm(vel, axis=-1, keepdims=True), charges], axis=-1)
    g0 = x_0 + x_0_batch[:, None] * 5
    g1 = x_1 + x_1_batch[:, None] * 5
    g2 = x_2 + x_2_batch[:, None] * 5
    x = {'0': feat_in[g0[:, 0]],
         '1': (feat_in[g1[:, 0]] + feat_in[g1[:, 1]]) / 2.0,
         '2': (feat_in[g2[:, 0]] + feat_in[g2[:, 1]] + feat_in[g2[:, 2]]) / 3.0}
    x = {d: f @ emb_W + emb_b for d, f in x.items()}
    adj = {'0_0': adj_0_0, '0_1': adj_0_1, '1_1': adj_1_1, '1_2': adj_1_2}
    adj['1_0'] = adj['0_1'][jnp.array([1, 0])]
    adj['2_1'] = adj['1_2'][jnp.array([1, 0])]
    sorted_adj = {}
    for at, e in adj.items():
        order = jnp.argsort(e[1])
        sorted_adj[at] = (e[0][order], e[1][order])
    adj = sorted_adj
    nseg = {'0': x['0'].shape[0], '1': x['1'].shape[0], '2': x['2'].shape[0]}
    needed = [(0, 1, 2)] * 5 + [(0, 1), (0,)]
    for l in range(L):
        new = {}
        for d in needed[l]:
            agg = x[str(d)] @ W_self[l, d]
            for ai, at in enumerate(ADJ_TYPES):
                s, t = at.split('_')
                if int(t) == d:
                    e0, e1 = adj[at]
                    seg = jax.ops.segment_sum(x[s][e0], e1,
                                              num_segments=nseg[str(d)],
                                              indices_are_sorted=True)
                    agg = agg + seg @ W_adj[l, ai]
            new[str(d)] = _silu(agg)
        x = new
    return _head(x['0'], loc, y.reshape(-1, 3),
                 pre_W1[0], pre_b1[0], pre_W2[0], pre_b2[0],
                 post_W1, post_b1, post_W2, post_b2)


# speedup vs baseline: 3.2952x; 3.1895x over previous
import functools

import jax
import jax.numpy as jnp
from jax import lax
from jax.experimental import pallas as pl
from jax.experimental.pallas import tpu as pltpu
from jax.experimental.pallas import tpu_sc as plsc

H = 64
L = 7
F32 = jnp.float32
I32 = jnp.int32

N0, N1, N2 = 50000, 100000, 100000
N0P, N1P, N2P = 50176, 100352, 100352
NPAD = {0: N0P, 1: N1P, 2: N2P}

WIN = 12500
GUARD = 8
ACC_ROWS = 12528
CLAMP_HI = 12520
CHUNK = 512

ADJ_CFG = [
    ('0_0', 0, 200000, N0, 0, 4, 0),
    ('1_0', 1, 200000, N0, 5, 4, 4),
    ('0_1', 0, 200000, N1, 10, 8, 8),
    ('1_1', 1, 400000, N1, 19, 8, 16),
    ('2_1', 2, 300000, N1, 28, 8, 24),
    ('1_2', 1, 300000, N1, 37, 8, 32),
]
OFFS_LEN = 48


def _silu(x):
    return x * lax.logistic(x)


def _sext(vec_ref, k):
    base = (k // 16) * 16
    v = vec_ref[pl.ds(base, 16)]
    lane = lax.iota(I32, 16)
    return jnp.max(jnp.where(lane == (k - base), v, jnp.int32(-2147483648)))


def _make_seg_kernel(active):
    cfgs = [c for c in ADJ_CFG if c[0] in active]
    mesh = plsc.VectorSubcoreMesh(core_axis_name="c", subcore_axis_name="s")
    out_type = [jax.ShapeDtypeStruct((NPAD[{N0: 0, N1: 1, N2: 2}[c[3]]], H), F32)
                for c in cfgs]

    def body(*refs):
        nin = 3 + 2 * len(cfgs) + 2
        xs = refs[0:3]
        edge_refs = refs[3:3 + 2 * len(cfgs)]
        offs_hbm = refs[3 + 2 * len(cfgs)]
        zeros_hbm = refs[3 + 2 * len(cfgs) + 1]
        g_refs = refs[nin:nin + len(cfgs)]
        (offs_v, zbuf, sidx, didx, rows, acc, isem, gsem) = refs[nin + len(cfgs):]

        cid = lax.axis_index("c")
        sid = lax.axis_index("s")
        pltpu.sync_copy(offs_hbm, offs_v)
        pltpu.sync_copy(zeros_hbm, zbuf)

        for ci, cfg in enumerate(cfgs):
            _, sdim, _, _, obase, nwin, gwbase = cfg
            x_ref = xs[sdim]
            srcs_ref = edge_refs[2 * ci]
            dsts_ref = edge_refs[2 * ci + 1]
            g_ref = g_refs[ci]

            def win_body(w, carry, x_ref=x_ref, srcs_ref=srcs_ref,
                         dsts_ref=dsts_ref, g_ref=g_ref, obase=obase,
                         gwbase=gwbase):
                own = (gwbase + w) % 2

                @pl.when(cid == own)
                def _():
                    wlo = w * WIN
                    k0 = obase + w
                    lo = _sext(offs_v, k0)
                    hi = _sext(offs_v, k0 + 1)
                    start = lo - lax.rem(lo, 8)
                    n = hi - start
                    nch = (n + (CHUNK - 1)) // CHUNK
                    zb = sid * 783
                    for i in range(6):
                        pltpu.sync_copy(zbuf, acc.at[pl.ds(zb + i * 128, 128)])
                    pltpu.sync_copy(zbuf.at[pl.ds(0, 15)],
                                    acc.at[pl.ds(zb + 768, 15)])
                    plsc.subcore_barrier()

                    nmine = (jnp.maximum(nch - sid, 0) + 15) // 16

                    def chunk_body(j, carry):
                        pos = pl.multiple_of(start + (sid + j * 16) * CHUNK, 8)
                        ids = []
                        for r in range(4):
                            ids.append(pltpu.async_copy(
                                srcs_ref.at[pl.ds(pos + r * 128, 128)],
                                sidx.at[r], isem))
                            ids.append(pltpu.async_copy(
                                dsts_ref.at[pl.ds(pos + r * 128, 128)],
                                didx.at[r], isem))
                        for d in ids:
                            d.wait()
                        gds = [pltpu.async_copy(x_ref.at[sidx.at[r]],
                                                rows.at[r], gsem)
                               for r in range(4)]
                        shift = wlo - GUARD
                        for r in range(4):
                            for t in range(8):
                                v = didx[r, pl.ds(t * 16, 16)]
                                v = jnp.clip(v - shift, 0, CLAMP_HI)
                                didx[r, pl.ds(t * 16, 16)] = v
                        for d in gds:
                            d.wait()
                        for r in range(4):
                            pltpu.sync_copy(rows.at[r], acc.at[didx.at[r]],
                                            add=True)
                        return carry

                    lax.fori_loop(0, nmine, chunk_body, 0)
                    plsc.subcore_barrier()
                    wb = sid * 781
                    pltpu.sync_copy(acc.at[pl.ds(GUARD + wb, 781)],
                                    g_ref.at[pl.ds(wlo + wb, 781)])

                    @pl.when(sid == 0)
                    def _tail():
                        pltpu.sync_copy(acc.at[pl.ds(GUARD + 12496, 4)],
                                        g_ref.at[pl.ds(wlo + 12496, 4)])
                    plsc.subcore_barrier()
                return carry

            lax.fori_loop(0, nwin, win_body, 0)

    return pl.kernel(
        body,
        out_type=out_type,
        mesh=mesh,
        compiler_params=pltpu.CompilerParams(use_tc_tiling_on_sc=False,
                                             needs_layout_passes=False),
        scratch_types=[
            pltpu.VMEM((OFFS_LEN,), I32),
            pltpu.VMEM((128, H), F32),
            pltpu.VMEM((4, 128), I32),
            pltpu.VMEM((4, 128), I32),
            pltpu.VMEM((4, 128, H), F32),
            pltpu.VMEM_SHARED((ACC_ROWS, H), F32),
            pltpu.SemaphoreType.DMA,
            pltpu.SemaphoreType.DMA,
        ],
    )


def _embed_gather_kernel():
    mesh = plsc.VectorSubcoreMesh(core_axis_name="c", subcore_axis_name="s")
    out_type = [
        jax.ShapeDtypeStruct((N0P, 16), F32),
        jax.ShapeDtypeStruct((N1P, 16), F32),
        jax.ShapeDtypeStruct((N2P, 16), F32),
    ]

    def body(featp, i0, i1a, i1b, i2a, i2b, i2c, f0, f1, f2,
             idxv, fa, fb, fc, gsem, isem):
        cid = lax.axis_index("c")
        sid = lax.axis_index("s")
        wid = sid * 2 + cid

        def run_dim(idx_refs, out_ref, nchunks, combine):
            nmine = (jnp.maximum(nchunks - wid, 0) + 31) // 32

            def chunk_body(j, carry):
                base = pl.multiple_of((wid + j * 32) * CHUNK, 8)
                bufs = [fa, fb, fc][:len(idx_refs)]
                for bi, iref in enumerate(idx_refs):
                    ids = []
                    for r in range(4):
                        ids.append(pltpu.async_copy(
                            iref.at[pl.ds(base + r * 128, 128)],
                            idxv.at[r], isem))
                    for d in ids:
                        d.wait()
                    gds = [pltpu.async_copy(featp.at[idxv.at[r]],
                                            bufs[bi].at[r], gsem)
                           for r in range(4)]
                    for d in gds:
                        d.wait()
                if combine:
                    for r in range(4):
                        def row_body(i, c, r=r):
                            v = fa[r, i]
                            v = v + fb[r, i]
                            if len(idx_refs) == 3:
                                v = v + fc[r, i]
                            fa[r, i] = v
                            return c
                        lax.fori_loop(0, 128, row_body, 0)
                for r in range(4):
                    pltpu.sync_copy(
                        fa.at[r], out_ref.at[pl.ds(base + r * 128, 128)])
                return carry

            lax.fori_loop(0, nmine, chunk_body, 0)

        run_dim([i0], f0, N0P // CHUNK, False)
        run_dim([i1a, i1b], f1, N1P // CHUNK, True)
        run_dim([i2a, i2b, i2c], f2, N2P // CHUNK, True)

    return pl.kernel(
        body,
        out_type=out_type,
        mesh=mesh,
        compiler_params=pltpu.CompilerParams(use_tc_tiling_on_sc=False),
        scratch_types=[
            pltpu.VMEM((4, 128), I32),
            pltpu.VMEM((4, 128, 16), F32),
            pltpu.VMEM((4, 128, 16), F32),
            pltpu.VMEM((4, 128, 16), F32),
            pltpu.SemaphoreType.DMA,
            pltpu.SemaphoreType.DMA,
        ],
    )


def _feat_body(vel_ref, ch_ref, out_ref):
    v = vel_ref[...]
    n = jnp.sqrt(jnp.sum(v * v, axis=1, keepdims=True))
    out_ref[...] = jnp.concatenate(
        [n, ch_ref[...], jnp.zeros((v.shape[0], 14), F32)], axis=1)


def _feat_kernel(vel, charges):
    blk = 2000
    return pl.pallas_call(
        _feat_body,
        grid=(N0 // blk,),
        in_specs=[pl.BlockSpec((blk, 3), lambda i: (i, 0)),
                  pl.BlockSpec((blk, 1), lambda i: (i, 0))],
        out_specs=pl.BlockSpec((blk, 16), lambda i: (i, 0)),
        out_shape=jax.ShapeDtypeStruct((N0, 16), F32),
    )(vel, charges)


def _emb_mm_body(f_ref, w_ref, b_ref, out_ref):
    out_ref[...] = jnp.dot(f_ref[...], w_ref[...],
                           preferred_element_type=F32) + b_ref[...]


def _emb_mm(f, w, b):
    npad = f.shape[0]
    blk = 1024
    return pl.pallas_call(
        _emb_mm_body,
        grid=(npad // blk,),
        in_specs=[pl.BlockSpec((blk, 16), lambda i: (i, 0)),
                  pl.BlockSpec((16, H), lambda i: (0, 0)),
                  pl.BlockSpec((1, H), lambda i: (0, 0))],
        out_specs=pl.BlockSpec((blk, H), lambda i: (i, 0)),
        out_shape=jax.ShapeDtypeStruct((npad, H), F32),
    )(f, w, b)


def _mk_mm_body(ng):
    def body(*refs):
        x_ref = refs[0]
        g_refs = refs[1:1 + ng]
        w_ref = refs[1 + ng]
        out_ref = refs[2 + ng]
        acc = jnp.dot(x_ref[...], w_ref[0], preferred_element_type=F32)
        for i, g in enumerate(g_refs):
            acc = acc + jnp.dot(g[...], w_ref[1 + i],
                                preferred_element_type=F32)
        out_ref[...] = _silu(acc)
    return body


def _layer_mm(x, gs, wstack):
    ng = len(gs)
    npad = x.shape[0]
    blk = 1024
    row = lambda i: (i, 0)
    fixed = lambda i: (0, 0, 0)
    return pl.pallas_call(
        _mk_mm_body(ng),
        grid=(npad // blk,),
        in_specs=[pl.BlockSpec((blk, H), row)]
        + [pl.BlockSpec((blk, H), row) for _ in range(ng)]
        + [pl.BlockSpec((ng + 1, H, H), fixed)],
        out_specs=pl.BlockSpec((blk, H), row),
        out_shape=jax.ShapeDtypeStruct((npad, H), F32),
    )(x, *gs, wstack)


HEAD_BLOCK = 2000


def _head_body(x0_ref, loc_ref, y_ref, pw1_ref, pb1_ref, pw2_ref, pb2_ref,
               postw1_ref, postb1_ref, postw2_ref, postb2_ref,
               loss_ref, lsum_ref, cnt_ref):
    x = x0_ref[...]
    h = _silu(x @ pw1_ref[...] + pb1_ref[...])
    h = h @ pw2_ref[...] + pb2_ref[...]
    o = _silu(h @ postw1_ref[...] + postb1_ref[...])
    o = o @ postw2_ref[...] + postb2_ref[...]
    o = o + loc_ref[...]
    t = y_ref[...]
    mask = ~jnp.isnan(t)
    diff2 = (o - t) ** 2
    loss = jnp.where(mask, diff2, 0.0)
    loss_ref[...] = loss

    @pl.when(pl.program_id(0) == 0)
    def _init():
        lsum_ref[...] = jnp.zeros_like(lsum_ref)
        cnt_ref[...] = jnp.zeros_like(cnt_ref)

    lsum_ref[...] += jnp.sum(loss).reshape(1, 1)
    cnt_ref[...] += jnp.sum(mask.astype(F32)).reshape(1, 1)


def _head(x0p, loc, y, pre_W1, pre_b1, pre_W2, pre_b2,
          post_W1, post_b1, post_W2, post_b2):
    grid = N0 // HEAD_BLOCK
    row = lambda i: (i, 0)
    fixed = lambda i: (0, 0)
    loss, lsum, cnt = pl.pallas_call(
        _head_body,
        grid=(grid,),
        in_specs=[
            pl.BlockSpec((HEAD_BLOCK, H), row),
            pl.BlockSpec((HEAD_BLOCK, 3), row),
            pl.BlockSpec((HEAD_BLOCK, 3), row),
            pl.BlockSpec((H, H), fixed),
            pl.BlockSpec((1, H), fixed),
            pl.BlockSpec((H, H), fixed),
            pl.BlockSpec((1, H), fixed),
            pl.BlockSpec((H, H), fixed),
            pl.BlockSpec((1, H), fixed),
            pl.BlockSpec((H, 3), fixed),
            pl.BlockSpec((1, 3), fixed),
        ],
        out_specs=[
            pl.BlockSpec((HEAD_BLOCK, 3), row),
            pl.BlockSpec((1, 1), fixed),
            pl.BlockSpec((1, 1), fixed),
        ],
        out_shape=[
            jax.ShapeDtypeStruct((N0, 3), F32),
            jax.ShapeDtypeStruct((1, 1), F32),
            jax.ShapeDtypeStruct((1, 1), F32),
        ],
    )(x0p, loc, y,
      pre_W1, pre_b1.reshape(1, H), pre_W2, pre_b2.reshape(1, H),
      post_W1, post_b1.reshape(1, H), post_W2, post_b2.reshape(1, 3))
    backprop = lsum[0, 0] / jnp.maximum(cnt[0, 0], 1.0)
    return backprop, loss


_SEG_FULL = _make_seg_kernel(['0_0', '1_0', '0_1', '1_1', '2_1', '1_2'])
_SEG_L5 = _make_seg_kernel(['0_0', '1_0', '0_1', '1_1', '2_1'])
_SEG_L6 = _make_seg_kernel(['0_0', '1_0'])
_EMB_GATHER = _embed_gather_kernel()

_TARGETS = {0: [0, 1], 1: [2, 3, 4], 2: [5]}


def _pad1(a, n, val):
    return jnp.pad(a.astype(I32), (0, n - a.shape[0]), constant_values=val)


def kernel(loc, vel, charges, y, x_0, x_0_batch, x_1, x_1_batch, x_2,
           x_2_batch, adj_0_0, adj_0_1, adj_1_1, adj_1_2, step, emb_W, emb_b,
           W_self, W_adj, pre_W1, pre_b1, pre_W2, pre_b2, post_W1, post_b1,
           post_W2, post_b2):
    i0 = _pad1(x_0[:, 0] + x_0_batch * 5, N0P, 0)
    i1a = _pad1(x_1[:, 0] + x_1_batch * 5, N1P, 0)
    i1b = _pad1(x_1[:, 1] + x_1_batch * 5, N1P, 0)
    i2a = _pad1(x_2[:, 0] + x_2_batch * 5, N2P, 0)
    i2b = _pad1(x_2[:, 1] + x_2_batch * 5, N2P, 0)
    i2c = _pad1(x_2[:, 2] + x_2_batch * 5, N2P, 0)

    raw_adj = {'0_0': (adj_0_0[0], adj_0_0[1]),
               '1_0': (adj_0_1[1], adj_0_1[0]),
               '0_1': (adj_0_1[0], adj_0_1[1]),
               '1_1': (adj_1_1[0], adj_1_1[1]),
               '2_1': (adj_1_2[1], adj_1_2[0]),
               '1_2': (adj_1_2[0], adj_1_2[1])}
    edges = {}
    offs_list = []
    for name, sdim, E, Nt, obase, nwin, _ in ADJ_CFG:
        src, dst = raw_adj[name]
        order = jnp.argsort(dst)
        srcs = _pad1(src[order], E + 1024, 0)
        dsts = _pad1(dst[order], E + 1024, Nt)
        edges[name] = (srcs, dsts)
        offs_list.append(jnp.searchsorted(
            dsts[:E], jnp.arange(nwin + 1, dtype=I32) * WIN).astype(I32))
    offs = jnp.concatenate(offs_list)
    offs = jnp.pad(offs, (0, OFFS_LEN - offs.shape[0]))
    zeros128 = jnp.zeros((128, H), F32)

    featp = _feat_kernel(vel, charges)
    f0, f1, f2 = _EMB_GATHER(featp, i0, i1a, i1b, i2a, i2b, i2c)
    embp = jnp.pad(emb_W.astype(F32), ((0, 14), (0, 0)))
    bias = emb_b.reshape(1, H).astype(F32)
    x = [_emb_mm(f0, embp, bias),
         _emb_mm(f1, embp * 0.5, bias),
         _emb_mm(f2, embp * (1.0 / 3.0), bias)]

    def seg_call(fn, names):
        args = [x[0], x[1], x[2]]
        for name, *_ in ADJ_CFG:
            if name in names:
                args += [edges[name][0], edges[name][1]]
        args += [offs, zeros128]
        return fn(*args)

    names_full = ['0_0', '1_0', '0_1', '1_1', '2_1', '1_2']
    names_l5 = ['0_0', '1_0', '0_1', '1_1', '2_1']
    names_l6 = ['0_0', '1_0']

    for l in range(L):
        if l < 5:
            g = seg_call(_SEG_FULL, names_full)
            gmap = dict(zip(names_full, g))
            dims = (0, 1, 2)
        elif l == 5:
            g = seg_call(_SEG_L5, names_l5)
            gmap = dict(zip(names_l5, g))
            dims = (0, 1)
        else:
            g = seg_call(_SEG_L6, names_l6)
            gmap = dict(zip(names_l6, g))
            dims = (0,)
        newx = list(x)
        for d in dims:
            ais = _TARGETS[d]
            gs = [gmap[names_full[ai]] for ai in ais]
            wstack = jnp.stack([W_self[l, d]] + [W_adj[l, ai] for ai in ais])
            newx[d] = _layer_mm(x[d], gs, wstack)
        x = newx

    return _head(x[0], loc, y.reshape(-1, 3),
                 pre_W1[0], pre_b1[0], pre_W2[0], pre_b2[0],
                 post_W1, post_b1, post_W2, post_b2)


# speedup vs baseline: 3.5583x; 1.0798x over previous
import functools

import jax
import jax.numpy as jnp
from jax import lax
from jax.experimental import pallas as pl
from jax.experimental.pallas import tpu as pltpu
from jax.experimental.pallas import tpu_sc as plsc

H = 64
L = 7
F32 = jnp.float32
I32 = jnp.int32

N0, N1, N2 = 50000, 100000, 100000
N0P, N1P, N2P = 50176, 100352, 100352
NPAD = {0: N0P, 1: N1P, 2: N2P}

WIN = 12500
GUARD = 8
ACC_ROWS = 12528
CLAMP_HI = 12520
CHUNK = 512

ADJ_CFG = [
    ('0_0', 0, 200000, N0, 0, 4, 0),
    ('1_0', 1, 200000, N0, 5, 4, 4),
    ('0_1', 0, 200000, N1, 10, 8, 8),
    ('1_1', 1, 400000, N1, 19, 8, 16),
    ('2_1', 2, 300000, N1, 28, 8, 24),
    ('1_2', 1, 300000, N1, 37, 8, 32),
]
OFFS_LEN = 48


def _silu(x):
    return x * lax.logistic(x)


def _sext(vec_ref, k):
    base = (k // 16) * 16
    v = vec_ref[pl.ds(base, 16)]
    lane = lax.iota(I32, 16)
    return jnp.max(jnp.where(lane == (k - base), v, jnp.int32(-2147483648)))


def _make_seg_kernel(active):
    cfgs = [c for c in ADJ_CFG if c[0] in active]
    mesh = plsc.VectorSubcoreMesh(core_axis_name="c", subcore_axis_name="s")
    out_type = [jax.ShapeDtypeStruct((NPAD[{N0: 0, N1: 1, N2: 2}[c[3]]], H), F32)
                for c in cfgs]

    def body(*refs):
        nin = 3 + 2 * len(cfgs) + 2
        xs = refs[0:3]
        edge_refs = refs[3:3 + 2 * len(cfgs)]
        offs_hbm = refs[3 + 2 * len(cfgs)]
        zeros_hbm = refs[3 + 2 * len(cfgs) + 1]
        g_refs = refs[nin:nin + len(cfgs)]
        (offs_v, zbuf, sidx, didx, rows, sidx2, didx2, rows2,
         acc, isem, gsem) = refs[nin + len(cfgs):]

        cid = lax.axis_index("c")
        sid = lax.axis_index("s")
        pltpu.sync_copy(offs_hbm, offs_v)
        pltpu.sync_copy(zeros_hbm, zbuf)

        for ci, cfg in enumerate(cfgs):
            _, sdim, _, _, obase, nwin, gwbase = cfg
            x_ref = xs[sdim]
            srcs_ref = edge_refs[2 * ci]
            dsts_ref = edge_refs[2 * ci + 1]
            g_ref = g_refs[ci]

            def win_body(w, carry, x_ref=x_ref, srcs_ref=srcs_ref,
                         dsts_ref=dsts_ref, g_ref=g_ref, obase=obase,
                         gwbase=gwbase):
                own = (gwbase + w) % 2

                @pl.when(cid == own)
                def _():
                    wlo = w * WIN
                    k0 = obase + w
                    lo = _sext(offs_v, k0)
                    hi = _sext(offs_v, k0 + 1)
                    start = lo - lax.rem(lo, 8)
                    n = hi - start
                    nch = (n + (CHUNK - 1)) // CHUNK
                    zb = sid * 783
                    for i in range(6):
                        pltpu.sync_copy(zbuf, acc.at[pl.ds(zb + i * 128, 128)])
                    pltpu.sync_copy(zbuf.at[pl.ds(0, 15)],
                                    acc.at[pl.ds(zb + 768, 15)])
                    plsc.subcore_barrier()

                    nmine = (jnp.maximum(nch - sid, 0) + 15) // 16
                    shift = wlo - GUARD

                    def issue_chunk(k, sx, dx, rw):
                        pos = pl.multiple_of(
                            start + (sid + k * 16) * CHUNK, 8)
                        ids = []
                        for r in range(4):
                            ids.append(pltpu.async_copy(
                                srcs_ref.at[pl.ds(pos + r * 128, 128)],
                                sx.at[r], isem))
                            ids.append(pltpu.async_copy(
                                dsts_ref.at[pl.ds(pos + r * 128, 128)],
                                dx.at[r], isem))
                        for d in ids:
                            d.wait()
                        for r in range(4):
                            for t in range(8):
                                v = dx[r, pl.ds(t * 16, 16)]
                                v = jnp.clip(v - shift, 0, CLAMP_HI)
                                dx[r, pl.ds(t * 16, 16)] = v
                        for r in range(4):
                            pltpu.async_copy(x_ref.at[sx.at[r]],
                                             rw.at[r], gsem)

                    def drain_scatter(dx, rw):
                        for r in range(4):
                            pltpu.make_async_copy(
                                x_ref.at[pl.ds(0, 128)], rw.at[r],
                                gsem).wait()
                        for r in range(4):
                            pltpu.sync_copy(rw.at[r], acc.at[dx.at[r]],
                                            add=True)

                    def chunk_body(j, carry):
                        k1 = 2 * j + 1
                        issue_chunk(2 * j, sidx, didx, rows)

                        @pl.when(k1 < nmine)
                        def _i1():
                            issue_chunk(k1, sidx2, didx2, rows2)
                        drain_scatter(didx, rows)

                        @pl.when(k1 < nmine)
                        def _s1():
                            drain_scatter(didx2, rows2)
                        return carry

                    lax.fori_loop(0, (nmine + 1) // 2, chunk_body, 0)
                    plsc.subcore_barrier()
                    wb = sid * 781
                    pltpu.sync_copy(acc.at[pl.ds(GUARD + wb, 781)],
                                    g_ref.at[pl.ds(wlo + wb, 781)])

                    @pl.when(sid == 0)
                    def _tail():
                        pltpu.sync_copy(acc.at[pl.ds(GUARD + 12496, 4)],
                                        g_ref.at[pl.ds(wlo + 12496, 4)])
                    plsc.subcore_barrier()
                return carry

            lax.fori_loop(0, nwin, win_body, 0)

    return pl.kernel(
        body,
        out_type=out_type,
        mesh=mesh,
        compiler_params=pltpu.CompilerParams(use_tc_tiling_on_sc=False,
                                             needs_layout_passes=False),
        scratch_types=[
            pltpu.VMEM((OFFS_LEN,), I32),
            pltpu.VMEM((128, H), F32),
            pltpu.VMEM((4, 128), I32),
            pltpu.VMEM((4, 128), I32),
            pltpu.VMEM((4, 128, H), F32),
            pltpu.VMEM((4, 128), I32),
            pltpu.VMEM((4, 128), I32),
            pltpu.VMEM((4, 128, H), F32),
            pltpu.VMEM_SHARED((ACC_ROWS, H), F32),
            pltpu.SemaphoreType.DMA,
            pltpu.SemaphoreType.DMA,
        ],
    )


def _embed_gather_kernel():
    mesh = plsc.VectorSubcoreMesh(core_axis_name="c", subcore_axis_name="s")
    out_type = [
        jax.ShapeDtypeStruct((N0P, 16), F32),
        jax.ShapeDtypeStruct((N1P, 16), F32),
        jax.ShapeDtypeStruct((N2P, 16), F32),
    ]

    def body(featp, i0, i1a, i1b, i2a, i2b, i2c, f0, f1, f2,
             idxv, fa, fb, fc, gsem, isem):
        cid = lax.axis_index("c")
        sid = lax.axis_index("s")
        wid = sid * 2 + cid

        def run_dim(idx_refs, out_ref, nchunks, combine):
            nmine = (jnp.maximum(nchunks - wid, 0) + 31) // 32

            def chunk_body(j, carry):
                base = pl.multiple_of((wid + j * 32) * CHUNK, 8)
                bufs = [fa, fb, fc][:len(idx_refs)]
                for bi, iref in enumerate(idx_refs):
                    ids = []
                    for r in range(4):
                        ids.append(pltpu.async_copy(
                            iref.at[pl.ds(base + r * 128, 128)],
                            idxv.at[r], isem))
                    for d in ids:
                        d.wait()
                    gds = [pltpu.async_copy(featp.at[idxv.at[r]],
                                            bufs[bi].at[r], gsem)
                           for r in range(4)]
                    for d in gds:
                        d.wait()
                if combine:
                    for r in range(4):
                        def row_body(i, c, r=r):
                            v = fa[r, i]
                            v = v + fb[r, i]
                            if len(idx_refs) == 3:
                                v = v + fc[r, i]
                            fa[r, i] = v
                            return c
                        lax.fori_loop(0, 128, row_body, 0)
                for r in range(4):
                    pltpu.sync_copy(
                        fa.at[r], out_ref.at[pl.ds(base + r * 128, 128)])
                return carry

            lax.fori_loop(0, nmine, chunk_body, 0)

        run_dim([i0], f0, N0P // CHUNK, False)
        run_dim([i1a, i1b], f1, N1P // CHUNK, True)
        run_dim([i2a, i2b, i2c], f2, N2P // CHUNK, True)

    return pl.kernel(
        body,
        out_type=out_type,
        mesh=mesh,
        compiler_params=pltpu.CompilerParams(use_tc_tiling_on_sc=False),
        scratch_types=[
            pltpu.VMEM((4, 128), I32),
            pltpu.VMEM((4, 128, 16), F32),
            pltpu.VMEM((4, 128, 16), F32),
            pltpu.VMEM((4, 128, 16), F32),
            pltpu.SemaphoreType.DMA,
            pltpu.SemaphoreType.DMA,
        ],
    )


def _feat_body(vel_ref, ch_ref, out_ref):
    v = vel_ref[...]
    n = jnp.sqrt(jnp.sum(v * v, axis=1, keepdims=True))
    out_ref[...] = jnp.concatenate(
        [n, ch_ref[...], jnp.zeros((v.shape[0], 14), F32)], axis=1)


def _feat_kernel(vel, charges):
    blk = 2000
    return pl.pallas_call(
        _feat_body,
        grid=(N0 // blk,),
        in_specs=[pl.BlockSpec((blk, 3), lambda i: (i, 0)),
                  pl.BlockSpec((blk, 1), lambda i: (i, 0))],
        out_specs=pl.BlockSpec((blk, 16), lambda i: (i, 0)),
        out_shape=jax.ShapeDtypeStruct((N0, 16), F32),
    )(vel, charges)


def _emb_mm_body(f_ref, w_ref, b_ref, out_ref):
    out_ref[...] = jnp.dot(f_ref[...], w_ref[...],
                           preferred_element_type=F32) + b_ref[...]


def _emb_mm(f, w, b):
    npad = f.shape[0]
    blk = 1024
    return pl.pallas_call(
        _emb_mm_body,
        grid=(npad // blk,),
        in_specs=[pl.BlockSpec((blk, 16), lambda i: (i, 0)),
                  pl.BlockSpec((16, H), lambda i: (0, 0)),
                  pl.BlockSpec((1, H), lambda i: (0, 0))],
        out_specs=pl.BlockSpec((blk, H), lambda i: (i, 0)),
        out_shape=jax.ShapeDtypeStruct((npad, H), F32),
    )(f, w, b)


def _mk_mm_body(ng):
    def body(*refs):
        x_ref = refs[0]
        g_refs = refs[1:1 + ng]
        w_ref = refs[1 + ng]
        out_ref = refs[2 + ng]
        acc = jnp.dot(x_ref[...], w_ref[0], preferred_element_type=F32)
        for i, g in enumerate(g_refs):
            acc = acc + jnp.dot(g[...], w_ref[1 + i],
                                preferred_element_type=F32)
        out_ref[...] = _silu(acc)
    return body


def _layer_mm(x, gs, wstack):
    ng = len(gs)
    npad = x.shape[0]
    blk = 1024
    row = lambda i: (i, 0)
    fixed = lambda i: (0, 0, 0)
    return pl.pallas_call(
        _mk_mm_body(ng),
        grid=(npad // blk,),
        in_specs=[pl.BlockSpec((blk, H), row)]
        + [pl.BlockSpec((blk, H), row) for _ in range(ng)]
        + [pl.BlockSpec((ng + 1, H, H), fixed)],
        out_specs=pl.BlockSpec((blk, H), row),
        out_shape=jax.ShapeDtypeStruct((npad, H), F32),
    )(x, *gs, wstack)


HEAD_BLOCK = 2000


def _head_body(x0_ref, loc_ref, y_ref, pw1_ref, pb1_ref, pw2_ref, pb2_ref,
               postw1_ref, postb1_ref, postw2_ref, postb2_ref,
               loss_ref, lsum_ref, cnt_ref):
    x = x0_ref[...]
    h = _silu(x @ pw1_ref[...] + pb1_ref[...])
    h = h @ pw2_ref[...] + pb2_ref[...]
    o = _silu(h @ postw1_ref[...] + postb1_ref[...])
    o = o @ postw2_ref[...] + postb2_ref[...]
    o = o + loc_ref[...]
    t = y_ref[...]
    mask = ~jnp.isnan(t)
    diff2 = (o - t) ** 2
    loss = jnp.where(mask, diff2, 0.0)
    loss_ref[...] = loss

    @pl.when(pl.program_id(0) == 0)
    def _init():
        lsum_ref[...] = jnp.zeros_like(lsum_ref)
        cnt_ref[...] = jnp.zeros_like(cnt_ref)

    lsum_ref[...] += jnp.sum(loss).reshape(1, 1)
    cnt_ref[...] += jnp.sum(mask.astype(F32)).reshape(1, 1)


def _head(x0p, loc, y, pre_W1, pre_b1, pre_W2, pre_b2,
          post_W1, post_b1, post_W2, post_b2):
    grid = N0 // HEAD_BLOCK
    row = lambda i: (i, 0)
    fixed = lambda i: (0, 0)
    loss, lsum, cnt = pl.pallas_call(
        _head_body,
        grid=(grid,),
        in_specs=[
            pl.BlockSpec((HEAD_BLOCK, H), row),
            pl.BlockSpec((HEAD_BLOCK, 3), row),
            pl.BlockSpec((HEAD_BLOCK, 3), row),
            pl.BlockSpec((H, H), fixed),
            pl.BlockSpec((1, H), fixed),
            pl.BlockSpec((H, H), fixed),
            pl.BlockSpec((1, H), fixed),
            pl.BlockSpec((H, H), fixed),
            pl.BlockSpec((1, H), fixed),
            pl.BlockSpec((H, 3), fixed),
            pl.BlockSpec((1, 3), fixed),
        ],
        out_specs=[
            pl.BlockSpec((HEAD_BLOCK, 3), row),
            pl.BlockSpec((1, 1), fixed),
            pl.BlockSpec((1, 1), fixed),
        ],
        out_shape=[
            jax.ShapeDtypeStruct((N0, 3), F32),
            jax.ShapeDtypeStruct((1, 1), F32),
            jax.ShapeDtypeStruct((1, 1), F32),
        ],
    )(x0p, loc, y,
      pre_W1, pre_b1.reshape(1, H), pre_W2, pre_b2.reshape(1, H),
      post_W1, post_b1.reshape(1, H), post_W2, post_b2.reshape(1, 3))
    backprop = lsum[0, 0] / jnp.maximum(cnt[0, 0], 1.0)
    return backprop, loss


_SEG_FULL = _make_seg_kernel(['0_0', '1_0', '0_1', '1_1', '2_1', '1_2'])
_SEG_L5 = _make_seg_kernel(['0_0', '1_0', '0_1', '1_1', '2_1'])
_SEG_L6 = _make_seg_kernel(['0_0', '1_0'])
_EMB_GATHER = _embed_gather_kernel()

_TARGETS = {0: [0, 1], 1: [2, 3, 4], 2: [5]}


def _pad1(a, n, val):
    return jnp.pad(a.astype(I32), (0, n - a.shape[0]), constant_values=val)


def kernel(loc, vel, charges, y, x_0, x_0_batch, x_1, x_1_batch, x_2,
           x_2_batch, adj_0_0, adj_0_1, adj_1_1, adj_1_2, step, emb_W, emb_b,
           W_self, W_adj, pre_W1, pre_b1, pre_W2, pre_b2, post_W1, post_b1,
           post_W2, post_b2):
    i0 = _pad1(x_0[:, 0] + x_0_batch * 5, N0P, 0)
    i1a = _pad1(x_1[:, 0] + x_1_batch * 5, N1P, 0)
    i1b = _pad1(x_1[:, 1] + x_1_batch * 5, N1P, 0)
    i2a = _pad1(x_2[:, 0] + x_2_batch * 5, N2P, 0)
    i2b = _pad1(x_2[:, 1] + x_2_batch * 5, N2P, 0)
    i2c = _pad1(x_2[:, 2] + x_2_batch * 5, N2P, 0)

    raw_adj = {'0_0': (adj_0_0[0], adj_0_0[1]),
               '1_0': (adj_0_1[1], adj_0_1[0]),
               '0_1': (adj_0_1[0], adj_0_1[1]),
               '1_1': (adj_1_1[0], adj_1_1[1]),
               '2_1': (adj_1_2[1], adj_1_2[0]),
               '1_2': (adj_1_2[0], adj_1_2[1])}
    edges = {}
    offs_list = []
    for name, sdim, E, Nt, obase, nwin, _ in ADJ_CFG:
        src, dst = raw_adj[name]
        dst_s, src_s = lax.sort_key_val(dst, src)
        srcs = _pad1(src_s, E + 1024, 0)
        dsts = _pad1(dst_s, E + 1024, Nt)
        edges[name] = (srcs, dsts)
        offs_list.append(jnp.searchsorted(
            dsts[:E], jnp.arange(nwin + 1, dtype=I32) * WIN).astype(I32))
    offs = jnp.concatenate(offs_list)
    offs = jnp.pad(offs, (0, OFFS_LEN - offs.shape[0]))
    zeros128 = jnp.zeros((128, H), F32)

    featp = _feat_kernel(vel, charges)
    f0, f1, f2 = _EMB_GATHER(featp, i0, i1a, i1b, i2a, i2b, i2c)
    embp = jnp.pad(emb_W.astype(F32), ((0, 14), (0, 0)))
    bias = emb_b.reshape(1, H).astype(F32)
    x = [_emb_mm(f0, embp, bias),
         _emb_mm(f1, embp * 0.5, bias),
         _emb_mm(f2, embp * (1.0 / 3.0), bias)]

    def seg_call(fn, names):
        args = [x[0], x[1], x[2]]
        for name, *_ in ADJ_CFG:
            if name in names:
                args += [edges[name][0], edges[name][1]]
        args += [offs, zeros128]
        return fn(*args)

    names_full = ['0_0', '1_0', '0_1', '1_1', '2_1', '1_2']
    names_l5 = ['0_0', '1_0', '0_1', '1_1', '2_1']
    names_l6 = ['0_0', '1_0']

    for l in range(L):
        if l < 5:
            g = seg_call(_SEG_FULL, names_full)
            gmap = dict(zip(names_full, g))
            dims = (0, 1, 2)
        elif l == 5:
            g = seg_call(_SEG_L5, names_l5)
            gmap = dict(zip(names_l5, g))
            dims = (0, 1)
        else:
            g = seg_call(_SEG_L6, names_l6)
            gmap = dict(zip(names_l6, g))
            dims = (0,)
        newx = list(x)
        for d in dims:
            ais = _TARGETS[d]
            gs = [gmap[names_full[ai]] for ai in ais]
            wstack = jnp.stack([W_self[l, d]] + [W_adj[l, ai] for ai in ais])
            newx[d] = _layer_mm(x[d], gs, wstack)
        x = newx

    return _head(x[0], loc, y.reshape(-1, 3),
                 pre_W1[0], pre_b1[0], pre_W2[0], pre_b2[0],
                 post_W1, post_b1, post_W2, post_b2)
